# baseline (speedup 1.0000x reference)
import functools

import jax
import jax.numpy as jnp
from jax import lax
from jax.experimental import pallas as pl
from jax.experimental.pallas import tpu as pltpu
from jax.experimental.pallas import tpu_sc as plsc

B = 16384
EMB = 32
N_TAB = 4
HID = 256
OUT = 20
SCENE = 20

NC, NS = 2, 16
NW = NC * NS
BPW = B // NW
CHUNK = 128
JPT = BPW // CHUNK


def _sc_gather(kid, vid, mid, sid, ktab, vtab, mtab, stab):
    mesh = plsc.VectorSubcoreMesh(core_axis_name="c", subcore_axis_name="s")

    @functools.partial(
        pl.kernel,
        mesh=mesh,
        compiler_params=pltpu.CompilerParams(use_tc_tiling_on_sc=False),
        out_type=jax.ShapeDtypeStruct((N_TAB, B, EMB), jnp.float32),
        scratch_types=[
            pltpu.VMEM((N_TAB * JPT, CHUNK), jnp.int32),
            pltpu.VMEM((N_TAB * BPW, EMB), jnp.float32),
            pltpu.VMEM_SHARED((86, EMB), jnp.float32),
            pltpu.VMEM_SHARED((86, EMB), jnp.float32),
            pltpu.VMEM_SHARED((24, EMB), jnp.float32),
            pltpu.VMEM_SHARED((8, EMB), jnp.float32),
            pltpu.SemaphoreType.DMA,
            pltpu.SemaphoreType.DMA,
            pltpu.SemaphoreType.DMA,
        ],
    )
    def k(kid_h, vid_h, mid_h, sid_h, kt_h, vt_h, mt_h, st_h, out_h,
          idx_v, rows_v, kt_s, vt_s, mt_s, st_s, sem_i, sem_g, sem_o):
        sid = lax.axis_index("s")
        wid = sid * NC + lax.axis_index("c")
        base = wid * BPW
        ids = (kid_h, vid_h, mid_h, sid_h)
        tabs = (kt_s, vt_s, mt_s, st_s)
        cps = [
            pltpu.async_copy(
                ids[t].at[pl.ds(base + CHUNK * j, CHUNK)],
                idx_v.at[t * JPT + j],
                sem_i,
            )
            for t in range(N_TAB)
            for j in range(JPT)
        ]
        @pl.when(sid == 0)
        def _stage_tables():
            for src, dst in zip((kt_h, vt_h, mt_h, st_h), tabs):
                pltpu.sync_copy(src, dst)
        for cp in cps:
            cp.wait()
        plsc.subcore_barrier()
        cps = [
            pltpu.async_copy(
                tabs[t].at[idx_v.at[t * JPT + j]],
                rows_v.at[pl.ds(BPW * t + CHUNK * j, CHUNK)],
                sem_g,
            )
            for t in range(N_TAB)
            for j in range(JPT)
        ]
        for cp in cps:
            cp.wait()
        cps = [
            pltpu.async_copy(
                rows_v.at[pl.ds(BPW * t, BPW)],
                out_h.at[t, pl.ds(base, BPW)],
                sem_o,
            )
            for t in range(N_TAB)
        ]
        for cp in cps:
            cp.wait()

    return k(kid, vid, mid, sid, ktab, vtab, mtab, stab)


def _mlp_body(g_ref, sc_ref, w1e_ref, w1s_ref, b1_ref, w2_ref, b2_ref, o_ref):
    h = jnp.dot(sc_ref[...], w1s_ref[...], preferred_element_type=jnp.float32)
    for t in range(N_TAB):
        h += jnp.dot(g_ref[t], w1e_ref[t], preferred_element_type=jnp.float32)
    h = jnp.maximum(h + b1_ref[...], 0.0)
    o_ref[...] = (
        jnp.dot(h, w2_ref[...], preferred_element_type=jnp.float32) + b2_ref[...]
    )


def _mlp(g, scene, w1e, w1s, b1, w2, b2, blk=2048):
    grid = B // blk
    return pl.pallas_call(
        _mlp_body,
        grid=(grid,),
        in_specs=[
            pl.BlockSpec((N_TAB, blk, EMB), lambda i: (0, i, 0)),
            pl.BlockSpec((blk, SCENE), lambda i: (i, 0)),
            pl.BlockSpec((N_TAB, EMB, HID), lambda i: (0, 0, 0)),
            pl.BlockSpec((SCENE, HID), lambda i: (0, 0)),
            pl.BlockSpec((1, HID), lambda i: (0, 0)),
            pl.BlockSpec((HID, OUT), lambda i: (0, 0)),
            pl.BlockSpec((1, OUT), lambda i: (0, 0)),
        ],
        out_specs=pl.BlockSpec((blk, OUT), lambda i: (i, 0)),
        out_shape=jax.ShapeDtypeStruct((B, OUT), jnp.float32),
    )(g, scene, w1e, w1s, b1.reshape(1, HID), w2, b2.reshape(1, OUT))


def kernel(killer_id, victim_id, move_id, stage_id, scene_tags,
           killer_table, victim_table, move_table, stage_table,
           W1, b1, W2, b2):
    g = _sc_gather(killer_id, victim_id, move_id, stage_id,
                   killer_table, victim_table, move_table, stage_table)
    w1e = W1[:N_TAB * EMB].reshape(N_TAB, EMB, HID)
    return _mlp(g, scene_tags, w1e, W1[N_TAB * EMB:], b1, W2, b2)


# speedup vs baseline: 1.5420x; 1.5420x over previous
import functools

import jax
import jax.numpy as jnp
from jax import lax
from jax.experimental import pallas as pl
from jax.experimental.pallas import tpu as pltpu
from jax.experimental.pallas import tpu_sc as plsc

B = 16384
EMB = 32
N_TAB = 4
HID = 256
OUT = 20
SCENE = 20

NC, NS = 2, 16
NW = NC * NS
BPW = B // NW
CHUNK = 128
JPT = BPW // CHUNK


def _sc_gather(kid, vid, mid, sid, ktab, vtab, mtab, stab):
    mesh = plsc.VectorSubcoreMesh(core_axis_name="c", subcore_axis_name="s")

    @functools.partial(
        pl.kernel,
        mesh=mesh,
        compiler_params=pltpu.CompilerParams(use_tc_tiling_on_sc=False),
        out_type=jax.ShapeDtypeStruct((B, N_TAB * EMB), jnp.float32),
        scratch_types=[
            pltpu.VMEM((N_TAB * JPT, CHUNK), jnp.int32),
            pltpu.VMEM((N_TAB * BPW, EMB), jnp.float32),
            pltpu.VMEM_SHARED((86, EMB), jnp.float32),
            pltpu.VMEM_SHARED((86, EMB), jnp.float32),
            pltpu.VMEM_SHARED((24, EMB), jnp.float32),
            pltpu.VMEM_SHARED((8, EMB), jnp.float32),
            pltpu.SemaphoreType.DMA,
            pltpu.SemaphoreType.DMA,
            pltpu.SemaphoreType.DMA,
        ],
    )
    def k(kid_h, vid_h, mid_h, sid_h, kt_h, vt_h, mt_h, st_h, out_h,
          idx_v, rows_v, kt_s, vt_s, mt_s, st_s, sem_i, sem_g, sem_o):
        sid = lax.axis_index("s")
        wid = sid * NC + lax.axis_index("c")
        base = wid * BPW
        ids = (kid_h, vid_h, mid_h, sid_h)
        tabs = (kt_s, vt_s, mt_s, st_s)
        cps = [
            pltpu.async_copy(
                ids[t].at[pl.ds(base + CHUNK * j, CHUNK)],
                idx_v.at[t * JPT + j],
                sem_i,
            )
            for t in range(N_TAB)
            for j in range(JPT)
        ]
        @pl.when(sid == 0)
        def _stage_tables():
            for src, dst in zip((kt_h, vt_h, mt_h, st_h), tabs):
                pltpu.sync_copy(src, dst)
        for cp in cps:
            cp.wait()
        plsc.subcore_barrier()
        cps = [
            pltpu.async_copy(
                tabs[t].at[idx_v.at[t * JPT + j]],
                rows_v.at[pl.ds(BPW * t + CHUNK * j, CHUNK)],
                sem_g,
            )
            for t in range(N_TAB)
            for j in range(JPT)
        ]
        for cp in cps:
            cp.wait()
        cps = [
            pltpu.async_copy(
                rows_v.at[pl.ds(BPW * t, BPW)],
                out_h.at[pl.ds(base, BPW), pl.ds(EMB * t, EMB)],
                sem_o,
            )
            for t in range(N_TAB)
        ]
        for cp in cps:
            cp.wait()

    return k(kid, vid, mid, sid, ktab, vtab, mtab, stab)


def _mlp_body(g_ref, sc_ref, w1e_ref, w1s_ref, b1_ref, w2_ref, b2_ref, o_ref):
    h = jnp.dot(sc_ref[...], w1s_ref[...], preferred_element_type=jnp.float32)
    h += jnp.dot(g_ref[...], w1e_ref[...], preferred_element_type=jnp.float32)
    h = jnp.maximum(h + b1_ref[...], 0.0)
    o_ref[...] = (
        jnp.dot(h, w2_ref[...], preferred_element_type=jnp.float32) + b2_ref[...]
    )


def _mlp(g, scene, w1e, w1s, b1, w2, b2, blk=2048):
    grid = B // blk
    return pl.pallas_call(
        _mlp_body,
        grid=(grid,),
        in_specs=[
            pl.BlockSpec((blk, N_TAB * EMB), lambda i: (i, 0)),
            pl.BlockSpec((blk, SCENE), lambda i: (i, 0)),
            pl.BlockSpec((N_TAB * EMB, HID), lambda i: (0, 0)),
            pl.BlockSpec((SCENE, HID), lambda i: (0, 0)),
            pl.BlockSpec((1, HID), lambda i: (0, 0)),
            pl.BlockSpec((HID, OUT), lambda i: (0, 0)),
            pl.BlockSpec((1, OUT), lambda i: (0, 0)),
        ],
        out_specs=pl.BlockSpec((blk, OUT), lambda i: (i, 0)),
        out_shape=jax.ShapeDtypeStruct((B, OUT), jnp.float32),
    )(g, scene, w1e, w1s, b1.reshape(1, HID), w2, b2.reshape(1, OUT))


def kernel(killer_id, victim_id, move_id, stage_id, scene_tags,
           killer_table, victim_table, move_table, stage_table,
           W1, b1, W2, b2):
    g = _sc_gather(killer_id, victim_id, move_id, stage_id,
                   killer_table, victim_table, move_table, stage_table)
    return _mlp(g, scene_tags, W1[:N_TAB * EMB], W1[N_TAB * EMB:], b1, W2, b2)


# speedup vs baseline: 1.5811x; 1.0253x over previous
import functools

import jax
import jax.numpy as jnp
from jax import lax
from jax.experimental import pallas as pl
from jax.experimental.pallas import tpu as pltpu
from jax.experimental.pallas import tpu_sc as plsc

B = 16384
EMB = 32
N_TAB = 4
HID = 256
OUT = 20
SCENE = 20

NC, NS = 2, 16
NW = NC * NS
BPW = B // NW
CHUNK = 128
JPT = BPW // CHUNK


def _sc_gather(kid, vid, mid, sid, ktab, vtab, mtab, stab):
    mesh = plsc.VectorSubcoreMesh(core_axis_name="c", subcore_axis_name="s")

    @functools.partial(
        pl.kernel,
        mesh=mesh,
        compiler_params=pltpu.CompilerParams(use_tc_tiling_on_sc=False),
        out_type=jax.ShapeDtypeStruct((B, N_TAB * EMB), jnp.float32),
        scratch_types=[
            pltpu.VMEM((N_TAB * JPT, CHUNK), jnp.int32),
            pltpu.VMEM((N_TAB * BPW, EMB), jnp.float32),
            pltpu.VMEM_SHARED((86, EMB), jnp.float32),
            pltpu.VMEM_SHARED((86, EMB), jnp.float32),
            pltpu.VMEM_SHARED((24, EMB), jnp.float32),
            pltpu.VMEM_SHARED((8, EMB), jnp.float32),
            pltpu.SemaphoreType.DMA,
            pltpu.SemaphoreType.DMA,
            pltpu.SemaphoreType.DMA,
        ],
    )
    def k(kid_h, vid_h, mid_h, sid_h, kt_h, vt_h, mt_h, st_h, out_h,
          idx_v, rows_v, kt_s, vt_s, mt_s, st_s, sem_i, sem_g, sem_o):
        sid = lax.axis_index("s")
        wid = sid * NC + lax.axis_index("c")
        base = wid * BPW
        ids = (kid_h, vid_h, mid_h, sid_h)
        tabs = (kt_s, vt_s, mt_s, st_s)
        cps = [
            pltpu.async_copy(
                ids[t].at[pl.ds(base + CHUNK * j, CHUNK)],
                idx_v.at[t * JPT + j],
                sem_i,
            )
            for t in range(N_TAB)
            for j in range(JPT)
        ]
        @pl.when(sid == 0)
        def _stage_tables():
            for src, dst in zip((kt_h, vt_h, mt_h, st_h), tabs):
                pltpu.sync_copy(src, dst)
        for cp in cps:
            cp.wait()
        plsc.subcore_barrier()
        cps = [
            pltpu.async_copy(
                tabs[t].at[idx_v.at[t * JPT + j]],
                rows_v.at[pl.ds(BPW * t + CHUNK * j, CHUNK)],
                sem_g,
            )
            for t in range(N_TAB)
            for j in range(JPT)
        ]
        for cp in cps:
            cp.wait()
        cps = [
            pltpu.async_copy(
                rows_v.at[pl.ds(BPW * t, BPW)],
                out_h.at[pl.ds(base, BPW), pl.ds(EMB * t, EMB)],
                sem_o,
            )
            for t in range(N_TAB)
        ]
        for cp in cps:
            cp.wait()

    return k(kid, vid, mid, sid, ktab, vtab, mtab, stab)


def _mlp_body(g_ref, sc_ref, w1_ref, b1_ref, w2_ref, b2_ref, o_ref):
    h = jnp.dot(sc_ref[...], w1_ref[pl.ds(N_TAB * EMB, SCENE), :],
                preferred_element_type=jnp.float32)
    h += jnp.dot(g_ref[...], w1_ref[pl.ds(0, N_TAB * EMB), :],
                 preferred_element_type=jnp.float32)
    h = jnp.maximum(h + b1_ref[...], 0.0)
    o_ref[...] = (
        jnp.dot(h, w2_ref[...], preferred_element_type=jnp.float32) + b2_ref[...]
    )


def _mlp(g, scene, w1, b1, w2, b2, blk=4096):
    grid = B // blk
    return pl.pallas_call(
        _mlp_body,
        grid=(grid,),
        in_specs=[
            pl.BlockSpec((blk, N_TAB * EMB), lambda i: (i, 0)),
            pl.BlockSpec((blk, SCENE), lambda i: (i, 0)),
            pl.BlockSpec((N_TAB * EMB + SCENE, HID), lambda i: (0, 0)),
            pl.BlockSpec((1, HID), lambda i: (0, 0)),
            pl.BlockSpec((HID, OUT), lambda i: (0, 0)),
            pl.BlockSpec((1, OUT), lambda i: (0, 0)),
        ],
        out_specs=pl.BlockSpec((blk, OUT), lambda i: (i, 0)),
        out_shape=jax.ShapeDtypeStruct((B, OUT), jnp.float32),
    )(g, scene, w1, b1.reshape(1, HID), w2, b2.reshape(1, OUT))


def kernel(killer_id, victim_id, move_id, stage_id, scene_tags,
           killer_table, victim_table, move_table, stage_table,
           W1, b1, W2, b2):
    g = _sc_gather(killer_id, victim_id, move_id, stage_id,
                   killer_table, victim_table, move_table, stage_table)
    return _mlp(g, scene_tags, W1, b1, W2, b2)


# speedup vs baseline: 2.0710x; 1.3099x over previous
import functools

import jax
import jax.numpy as jnp
from jax import lax
from jax.experimental import pallas as pl
from jax.experimental.pallas import tpu as pltpu
from jax.experimental.pallas import tpu_sc as plsc

B = 16384
EMB = 32
N_TAB = 4
HID = 256
OUT = 20
SCENE = 20

NC, NS = 2, 16
NW = NC * NS
BPW = B // NW
CHUNK = 128
JPT = BPW // CHUNK


def _sc_gather(kid, vid, mid, sid, ktab, vtab, mtab, stab):
    mesh = plsc.VectorSubcoreMesh(core_axis_name="c", subcore_axis_name="s")

    @functools.partial(
        pl.kernel,
        mesh=mesh,
        compiler_params=pltpu.CompilerParams(use_tc_tiling_on_sc=False),
        out_type=jax.ShapeDtypeStruct((B, N_TAB * EMB), jnp.float32),
        scratch_types=[
            pltpu.VMEM((N_TAB, JPT, CHUNK), jnp.int32),
            pltpu.VMEM((N_TAB * BPW, EMB), jnp.float32),
            pltpu.VMEM_SHARED((86, EMB), jnp.float32),
            pltpu.VMEM_SHARED((86, EMB), jnp.float32),
            pltpu.VMEM_SHARED((24, EMB), jnp.float32),
            pltpu.VMEM_SHARED((8, EMB), jnp.float32),
            pltpu.SemaphoreType.DMA,
            pltpu.SemaphoreType.DMA,
            pltpu.SemaphoreType.DMA,
        ],
    )
    def k(kid_h, vid_h, mid_h, sid_h, kt_h, vt_h, mt_h, st_h, out_h,
          idx_v, rows_v, kt_s, vt_s, mt_s, st_s, sem_i, sem_g, sem_o):
        sid_ax = lax.axis_index("s")
        wid = sid_ax * NC + lax.axis_index("c")
        base = wid * BPW
        ids = (kid_h, vid_h, mid_h, sid_h)
        tabs = (kt_s, vt_s, mt_s, st_s)
        cps = [
            pltpu.async_copy(ids[t].at[wid], idx_v.at[t], sem_i)
            for t in range(N_TAB)
        ]
        @pl.when(sid_ax == 0)
        def _stage_tables():
            for src, dst in zip((kt_h, vt_h, mt_h, st_h), tabs):
                pltpu.sync_copy(src, dst)
        for cp in cps:
            cp.wait()
        plsc.subcore_barrier()
        cps = [
            pltpu.async_copy(
                tabs[t].at[idx_v.at[t, j]],
                rows_v.at[pl.ds(BPW * t + CHUNK * j, CHUNK)],
                sem_g,
            )
            for t in range(N_TAB)
            for j in range(JPT)
        ]
        for cp in cps:
            cp.wait()
        cps = [
            pltpu.async_copy(
                rows_v.at[pl.ds(BPW * t, BPW)],
                out_h.at[pl.ds(base, BPW), pl.ds(EMB * t, EMB)],
                sem_o,
            )
            for t in range(N_TAB)
        ]
        for cp in cps:
            cp.wait()

    return k(kid, vid, mid, sid, ktab, vtab, mtab, stab)


def _dot(a, b, dims):
    return lax.dot_general(a, b, dimension_numbers=(dims, ((), ())),
                           preferred_element_type=jnp.float32)


def _mlp_body(g_ref, scT_ref, w1T_ref, b1_ref, w2T_ref, b2_ref, o_ref):
    hT = _dot(w1T_ref[:, pl.ds(0, N_TAB * EMB)], g_ref[...], ((1,), (1,)))
    hT += _dot(w1T_ref[:, pl.ds(N_TAB * EMB, SCENE)], scT_ref[...], ((1,), (0,)))
    hT = jnp.maximum(hT + b1_ref[...], 0.0)
    o_ref[...] = _dot(w2T_ref[...], hT, ((1,), (0,))) + b2_ref[...]


def _mlp_t(g, sceneT, w1T, b1c, w2T, b2c, blk=4096):
    grid = B // blk
    outT = pl.pallas_call(
        _mlp_body,
        grid=(grid,),
        in_specs=[
            pl.BlockSpec((blk, N_TAB * EMB), lambda i: (i, 0)),
            pl.BlockSpec((SCENE, blk), lambda i: (0, i)),
            pl.BlockSpec((HID, N_TAB * EMB + SCENE), lambda i: (0, 0)),
            pl.BlockSpec((HID, 1), lambda i: (0, 0)),
            pl.BlockSpec((OUT, HID), lambda i: (0, 0)),
            pl.BlockSpec((OUT, 1), lambda i: (0, 0)),
        ],
        out_specs=pl.BlockSpec((OUT, blk), lambda i: (0, i)),
        out_shape=jax.ShapeDtypeStruct((OUT, B), jnp.float32),
    )(g, sceneT, w1T, b1c, w2T, b2c)
    return outT


def kernel(killer_id, victim_id, move_id, stage_id, scene_tags,
           killer_table, victim_table, move_table, stage_table,
           W1, b1, W2, b2):
    ids3 = [i.reshape(NW, JPT, CHUNK) for i in
            (killer_id, victim_id, move_id, stage_id)]
    g = _sc_gather(*ids3, killer_table, victim_table, move_table, stage_table)
    outT = _mlp_t(g, scene_tags.T, W1.T, b1.reshape(HID, 1),
                  W2.T, b2.reshape(OUT, 1))
    return outT.T


# speedup vs baseline: 2.1090x; 1.0184x over previous
import functools

import jax
import jax.numpy as jnp
from jax import lax
from jax.experimental import pallas as pl
from jax.experimental.pallas import tpu as pltpu
from jax.experimental.pallas import tpu_sc as plsc

B = 16384
EMB = 32
N_TAB = 4
HID = 256
OUT = 20
SCENE = 20

NC, NS = 2, 16
NW = NC * NS
BPW = B // NW
CHUNK = 128
JPT = BPW // CHUNK


TAB_ROWS = 86 + 86 + 24 + 8


def _sc_gather(ids_r, ctab):
    mesh = plsc.VectorSubcoreMesh(core_axis_name="c", subcore_axis_name="s")

    @functools.partial(
        pl.kernel,
        mesh=mesh,
        compiler_params=pltpu.CompilerParams(use_tc_tiling_on_sc=False),
        out_type=jax.ShapeDtypeStruct((B, N_TAB * EMB), jnp.float32),
        scratch_types=[
            pltpu.VMEM((N_TAB, JPT, CHUNK), jnp.int32),
            pltpu.VMEM((N_TAB * BPW, EMB), jnp.float32),
            pltpu.VMEM_SHARED((TAB_ROWS, EMB), jnp.float32),
            pltpu.SemaphoreType.DMA,
            pltpu.SemaphoreType.DMA,
            pltpu.SemaphoreType.DMA,
        ],
    )
    def k(ids_h, ctab_h, out_h, idx_v, rows_v, ctab_s, sem_i, sem_g, sem_o):
        sid_ax = lax.axis_index("s")
        wid = sid_ax * NC + lax.axis_index("c")
        base = wid * BPW
        cps = [
            pltpu.async_copy(ids_h.at[t, wid], idx_v.at[t], sem_i)
            for t in range(N_TAB)
        ]
        @pl.when(sid_ax == 0)
        def _stage_tables():
            pltpu.sync_copy(ctab_h, ctab_s)
        for cp in cps:
            cp.wait()
        plsc.subcore_barrier()
        cps = [
            pltpu.async_copy(
                ctab_s.at[idx_v.at[t, j]],
                rows_v.at[pl.ds(BPW * t + CHUNK * j, CHUNK)],
                sem_g,
            )
            for t in range(N_TAB)
            for j in range(JPT)
        ]
        for cp in cps:
            cp.wait()
        cps = [
            pltpu.async_copy(
                rows_v.at[pl.ds(BPW * t, BPW)],
                out_h.at[pl.ds(base, BPW), pl.ds(EMB * t, EMB)],
                sem_o,
            )
            for t in range(N_TAB)
        ]
        for cp in cps:
            cp.wait()

    return k(ids_r, ctab)


def _dot(a, b, dims):
    return lax.dot_general(a, b, dimension_numbers=(dims, ((), ())),
                           preferred_element_type=jnp.float32)


def _mlp_body(g_ref, scT_ref, w1T_ref, b1_ref, w2T_ref, b2_ref, o_ref):
    hT = _dot(w1T_ref[:, pl.ds(0, N_TAB * EMB)], g_ref[...], ((1,), (1,)))
    hT += _dot(w1T_ref[:, pl.ds(N_TAB * EMB, SCENE)], scT_ref[...], ((1,), (0,)))
    hT = jnp.maximum(hT + b1_ref[...], 0.0)
    o_ref[...] = _dot(w2T_ref[...], hT, ((1,), (0,))) + b2_ref[...]


def _mlp_t(g, sceneT, w1T, b1c, w2T, b2c, blk=4096):
    grid = B // blk
    outT = pl.pallas_call(
        _mlp_body,
        grid=(grid,),
        in_specs=[
            pl.BlockSpec((blk, N_TAB * EMB), lambda i: (i, 0)),
            pl.BlockSpec((SCENE, blk), lambda i: (0, i)),
            pl.BlockSpec((HID, N_TAB * EMB + SCENE), lambda i: (0, 0)),
            pl.BlockSpec((HID, 1), lambda i: (0, 0)),
            pl.BlockSpec((OUT, HID), lambda i: (0, 0)),
            pl.BlockSpec((OUT, 1), lambda i: (0, 0)),
        ],
        out_specs=pl.BlockSpec((OUT, blk), lambda i: (0, i)),
        out_shape=jax.ShapeDtypeStruct((OUT, B), jnp.float32),
    )(g, sceneT, w1T, b1c, w2T, b2c)
    return outT


def kernel(killer_id, victim_id, move_id, stage_id, scene_tags,
           killer_table, victim_table, move_table, stage_table,
           W1, b1, W2, b2):
    ids_r = jnp.concatenate(
        [killer_id, victim_id + 86, move_id + 172, stage_id + 196]
    ).reshape(N_TAB, NW, JPT, CHUNK)
    ctab = jnp.concatenate(
        [killer_table, victim_table, move_table, stage_table], axis=0
    )
    g = _sc_gather(ids_r, ctab)
    outT = _mlp_t(g, scene_tags.T, W1.T, b1.reshape(HID, 1),
                  W2.T, b2.reshape(OUT, 1))
    return outT.T
